# baseline (speedup 1.0000x reference)
import dataclasses
import functools

import jax
import jax.numpy as jnp
import numpy as np
from jax import lax
from jax.experimental import pallas as pl
from jax.experimental.pallas import tpu as pltpu
from jax.experimental.pallas import tpu_sc as plsc

_WIN = 128


def _worker_range(mesh, nwin):
    wid = lax.axis_index("s") * mesh.num_cores + lax.axis_index("c")
    NW = mesh.num_cores * mesh.num_subcores
    base, rem = divmod(nwin, NW)
    lo = wid * base + jnp.minimum(wid, rem)
    hi = lo + base + jnp.where(wid < rem, 1, 0)
    return lo, hi


def _sc_gather_adj(table, adjm, nb):
    T, D = table.shape
    R2, Mw = adjm.shape
    E = R2 * Mw
    nwin = E // _WIN
    rpw = _WIN // Mw
    mesh = plsc.VectorSubcoreMesh(core_axis_name="c", subcore_axis_name="s")
    NW = mesh.num_cores * mesh.num_subcores
    NB = 4
    tmax = (nwin // NW + 1 + NB - 1) // NB

    @functools.partial(
        pl.kernel,
        out_type=jax.ShapeDtypeStruct((E, D), table.dtype),
        mesh=mesh,
        scratch_types=[
            pltpu.VMEM((NB, rpw, Mw), jnp.int32),
            pltpu.VMEM((NB, _WIN), jnp.int32),
            pltpu.VMEM((NB, _WIN, D), table.dtype),
            pltpu.SemaphoreType.DMA((NB,)),
            pltpu.SemaphoreType.DMA((NB,)),
            pltpu.SemaphoreType.DMA((NB,)),
        ],
    )
    def k(table_hbm, adj_hbm, out_hbm, idx_raw, idx_flat, rows_v,
          sem_i, sem_g, sem_w):
        lo, hi = _worker_range(mesh, nwin)

        def idx_copy(w, b):
            return pltpu.make_async_copy(
                adj_hbm.at[pl.ds(w * rpw, rpw), :], idx_raw.at[b], sem_i.at[b])

        def gather(b):
            return pltpu.make_async_copy(
                table_hbm.at[idx_flat.at[b]], rows_v.at[b], sem_g.at[b])

        def writeout(w, b):
            return pltpu.make_async_copy(
                rows_v.at[b], out_hbm.at[pl.ds(w * _WIN, _WIN)], sem_w.at[b])

        for b in range(NB):
            w = lo + b

            @pl.when(w < hi)
            def _():
                idx_copy(w, b).start()

        @pl.loop(0, tmax)
        def _(t):
            for b in range(NB):
                w = lo + t * NB + b

                @pl.when(w < hi)
                def _():
                    @pl.when(t > 0)
                    def _():
                        writeout(w, b).wait()

                    idx_copy(w, b).wait()
                    off = ((w * rpw) // nb) * nb
                    for r in range(rpw):
                        idx_flat[b, pl.ds(r * Mw, Mw)] = idx_raw[b, r, :] + off
                    gather(b).start()

            for b in range(NB):
                w = lo + t * NB + b

                @pl.when(w < hi)
                def _():
                    gather(b).wait()
                    nw = w + NB

                    @pl.when(nw < hi)
                    def _():
                        idx_copy(nw, b).start()

                    writeout(w, b).start()

        for b in range(NB):
            writeout(lo, b).wait()

    return k(table, adjm)


def _sc_gather_pairs(table, pairs, nmb, nb):
    T, D = table.shape
    E = pairs.shape[0]
    nwin = E // _WIN
    mesh = plsc.VectorSubcoreMesh(core_axis_name="c", subcore_axis_name="s")
    NW = mesh.num_cores * mesh.num_subcores
    NB = 2
    tmax = (nwin // NW + 1 + NB - 1) // NB
    L = 16
    cp = pltpu.CompilerParams()
    if "needs_layout_passes" in pltpu.CompilerParams.__dataclass_fields__:
        cp = dataclasses.replace(cp, needs_layout_passes=False)

    @functools.partial(
        pl.kernel,
        out_type=[jax.ShapeDtypeStruct((E, D), table.dtype),
                  jax.ShapeDtypeStruct((E, D), table.dtype)],
        mesh=mesh,
        compiler_params=cp,
        scratch_types=[
            pltpu.VMEM((NB, _WIN, 2), jnp.int32),
            pltpu.VMEM((NB, _WIN), jnp.int32),
            pltpu.VMEM((NB, _WIN), jnp.int32),
            pltpu.VMEM((NB, _WIN, D), table.dtype),
            pltpu.VMEM((NB, _WIN, D), table.dtype),
            pltpu.SemaphoreType.DMA((NB,)),
            pltpu.SemaphoreType.DMA((NB,)),
            pltpu.SemaphoreType.DMA((NB,)),
            pltpu.SemaphoreType.DMA((NB,)),
            pltpu.SemaphoreType.DMA((NB,)),
        ],
    )
    def k(table_hbm, pairs_hbm, d0_hbm, d1_hbm, idx_raw, f0, f1, r0, r1,
          sem_i, sg0, sg1, sw0, sw1):
        lo, hi = _worker_range(mesh, nwin)

        def idx_copy(w, b):
            return pltpu.make_async_copy(
                pairs_hbm.at[pl.ds(w * _WIN, _WIN), :], idx_raw.at[b],
                sem_i.at[b])

        def gather0(b):
            return pltpu.make_async_copy(
                table_hbm.at[f0.at[b]], r0.at[b], sg0.at[b])

        def gather1(b):
            return pltpu.make_async_copy(
                table_hbm.at[f1.at[b]], r1.at[b], sg1.at[b])

        def wout0(w, b):
            return pltpu.make_async_copy(
                r0.at[b], d0_hbm.at[pl.ds(w * _WIN, _WIN)], sw0.at[b])

        def wout1(w, b):
            return pltpu.make_async_copy(
                r1.at[b], d1_hbm.at[pl.ds(w * _WIN, _WIN)], sw1.at[b])

        for b in range(NB):
            w = lo + b

            @pl.when(w < hi)
            def _():
                idx_copy(w, b).start()

        @pl.loop(0, tmax)
        def _(t):
            for b in range(NB):
                w = lo + t * NB + b

                @pl.when(w < hi)
                def _():
                    @pl.when(t > 0)
                    def _():
                        wout0(w, b).wait()
                        wout1(w, b).wait()

                    idx_copy(w, b).wait()
                    off = ((w * _WIN) // nmb) * nb
                    lanes0 = jnp.zeros((L,), jnp.int32)
                    lanes1 = lanes0 + 1
                    for g in range(_WIN // L):
                        rows16 = lax.iota(jnp.int32, L) + g * L
                        f0[b, pl.ds(g * L, L)] = (
                            plsc.load_gather(idx_raw.at[b], [rows16, lanes0])
                            + off)
                        f1[b, pl.ds(g * L, L)] = (
                            plsc.load_gather(idx_raw.at[b], [rows16, lanes1])
                            + off)
                    gather0(b).start()
                    gather1(b).start()

            for b in range(NB):
                w = lo + t * NB + b

                @pl.when(w < hi)
                def _():
                    gather0(b).wait()
                    gather1(b).wait()
                    nw = w + NB

                    @pl.when(nw < hi)
                    def _():
                        idx_copy(nw, b).start()

                    wout0(w, b).start()
                    wout1(w, b).start()

        for b in range(NB):
            wout0(lo, b).wait()
            wout1(lo, b).wait()

    return k(table, pairs)


def _k1_body(atom_ref, bond_ref, sel_ref, r_ref, w_ref):
    a = atom_ref[...]
    r_ref[...] = jnp.sqrt(jnp.abs(a))
    sq = bond_ref[...]
    ssq = jnp.dot(sq * sq, sel_ref[...], preferred_element_type=jnp.float32)
    wun = 1.0 / ssq
    den = jnp.maximum(jnp.sum(wun, axis=-1, keepdims=True), 1e-12)
    w_ref[...] = wun / den


def _k3_body(nblk, m, g_ref, w_ref, r_ref, wn_ref, bn_ref, au_ref):
    g = g_ref[...].reshape(nblk, m, g_ref.shape[-1])
    w = w_ref[...]
    anw = jnp.sum(g * w[:, :, None], axis=1)
    x = r_ref[...] * anw
    y = jnp.dot(x, wn_ref[...], preferred_element_type=jnp.float32) + bn_ref[...]
    au_ref[...] = jnp.maximum(y, 0.0)


def _k4_body(nj, d0_ref, d1_ref, s_ref, acc_ref):
    j = pl.program_id(1)

    @pl.when(j == 0)
    def _():
        acc_ref[...] = jnp.zeros_like(acc_ref)

    c0 = jnp.sum(d0_ref[...], axis=0, keepdims=True)
    c1 = jnp.sum(d1_ref[...], axis=0, keepdims=True)
    acc_ref[...] += jnp.concatenate([c0, c1], axis=0)

    @pl.when(j == nj - 1)
    def _():
        s_ref[...] = acc_ref[...].reshape(s_ref.shape)


def _k6_body(nbb, bond_ref, d0_ref, d1_ref, s_ref, wt_ref, wb_ref, we_ref,
             bnte_ref, bedge_ref, out_ref):
    b = pl.program_id(0) // nbb
    s0 = s_ref[pl.ds(2 * b, 1), :]
    s1 = s_ref[pl.ds(2 * b + 1, 1), :]
    r0 = 1.0 / jnp.maximum(s0, 1e-12)
    r1 = 1.0 / jnp.maximum(s1, 1e-12)
    t = jnp.dot(d0_ref[...] * r0, wt_ref[...], preferred_element_type=jnp.float32)
    t += jnp.dot(d1_ref[...] * r1, wb_ref[...], preferred_element_type=jnp.float32)
    y = jnp.tanh(t + bnte_ref[...])
    z = bond_ref[...] + y
    out_ref[...] = (
        jnp.dot(z, we_ref[...], preferred_element_type=jnp.float32) + bedge_ref[...]
    )


def kernel(atom, bond, adj_matrix, adj_matrix_tuple, weight_node, weight_edge,
           weight_node_to_edge, bias_node, bias_edge, bias_node_to_edge):
    B, N, Fa = atom.shape
    M = adj_matrix.shape[-1]
    Fb = bond.shape[-1]
    BN = B * N
    NM = N * M
    TE = B * NM
    f32 = jnp.float32

    atom2 = atom.reshape(BN, Fa)
    bondf = bond.reshape(BN, M * Fb)
    sel = jnp.asarray(np.repeat(np.eye(M, dtype=np.float32), Fb, axis=0))

    blk1 = 2000
    R, w = pl.pallas_call(
        _k1_body,
        grid=(BN // blk1,),
        in_specs=[
            pl.BlockSpec((blk1, Fa), lambda i: (i, 0)),
            pl.BlockSpec((blk1, M * Fb), lambda i: (i, 0)),
            pl.BlockSpec((M * Fb, M), lambda i: (0, 0)),
        ],
        out_specs=[
            pl.BlockSpec((blk1, Fa), lambda i: (i, 0)),
            pl.BlockSpec((blk1, M), lambda i: (i, 0)),
        ],
        out_shape=[
            jax.ShapeDtypeStruct((BN, Fa), f32),
            jax.ShapeDtypeStruct((BN, M), f32),
        ],
    )(atom2, bondf, sel)

    G = _sc_gather_adj(R, adj_matrix.reshape(BN, M), N)

    blk3 = 400
    au2 = pl.pallas_call(
        functools.partial(_k3_body, blk3, M),
        grid=(BN // blk3,),
        in_specs=[
            pl.BlockSpec((blk3 * M, Fa), lambda i: (i, 0)),
            pl.BlockSpec((blk3, M), lambda i: (i, 0)),
            pl.BlockSpec((blk3, Fa), lambda i: (i, 0)),
            pl.BlockSpec((Fa, Fa), lambda i: (0, 0)),
            pl.BlockSpec((1, Fa), lambda i: (0, 0)),
        ],
        out_specs=pl.BlockSpec((blk3, Fa), lambda i: (i, 0)),
        out_shape=jax.ShapeDtypeStruct((BN, Fa), f32),
    )(G, w, R, weight_node, bias_node.reshape(1, Fa))

    D0, D1 = _sc_gather_pairs(au2, adj_matrix_tuple.reshape(TE, 2), NM, N)

    blk4 = 256
    nj = NM // blk4
    s = pl.pallas_call(
        functools.partial(_k4_body, nj),
        grid=(B, nj),
        in_specs=[
            pl.BlockSpec((blk4, Fa), lambda b, j: (b * nj + j, 0)),
            pl.BlockSpec((blk4, Fa), lambda b, j: (b * nj + j, 0)),
        ],
        out_specs=pl.BlockSpec((1, 2, Fa), lambda b, j: (b, 0, 0)),
        out_shape=jax.ShapeDtypeStruct((B, 2, Fa), f32),
        scratch_shapes=[pltpu.VMEM((2, Fa), f32)],
    )(D0, D1)
    s = s.reshape(2 * B, Fa)

    blk6 = 2000
    nb6 = TE // blk6
    nbb = NM // blk6
    outE = pl.pallas_call(
        functools.partial(_k6_body, nbb),
        grid=(nb6,),
        in_specs=[
            pl.BlockSpec((blk6, Fb), lambda i: (i, 0)),
            pl.BlockSpec((blk6, Fa), lambda i: (i, 0)),
            pl.BlockSpec((blk6, Fa), lambda i: (i, 0)),
            pl.BlockSpec((2 * B, Fa), lambda i: (0, 0)),
            pl.BlockSpec((Fa, Fb), lambda i: (0, 0)),
            pl.BlockSpec((Fa, Fb), lambda i: (0, 0)),
            pl.BlockSpec((Fb, Fb), lambda i: (0, 0)),
            pl.BlockSpec((1, Fb), lambda i: (0, 0)),
            pl.BlockSpec((1, Fb), lambda i: (0, 0)),
        ],
        out_specs=pl.BlockSpec((blk6, Fb), lambda i: (i, 0)),
        out_shape=jax.ShapeDtypeStruct((TE, Fb), f32),
    )(bond.reshape(TE, Fb), D0, D1, s,
      weight_node_to_edge[:Fa], weight_node_to_edge[Fa:], weight_edge,
      bias_node_to_edge.reshape(1, Fb), bias_edge.reshape(1, Fb))

    return (au2.reshape(B, N, Fa), outE.reshape(B, N, M, Fb))


# speedup vs baseline: 1.0603x; 1.0603x over previous
import functools

import jax
import jax.numpy as jnp
from jax import lax
from jax.experimental import pallas as pl
from jax.experimental.pallas import tpu as pltpu
from jax.experimental.pallas import tpu_sc as plsc

_WIN = 128
_NBUF = 4


def _sc_gather(table, idx):
    T, D = table.shape
    E = idx.shape[0]
    nwin = E // _WIN
    mesh = plsc.VectorSubcoreMesh(core_axis_name="c", subcore_axis_name="s")
    NW = mesh.num_cores * mesh.num_subcores
    base, rem = divmod(nwin, NW)
    tmax = (base + 1 + _NBUF - 1) // _NBUF

    @functools.partial(
        pl.kernel,
        out_type=jax.ShapeDtypeStruct((E, D), table.dtype),
        mesh=mesh,
        scratch_types=[
            pltpu.VMEM((_NBUF, _WIN), jnp.int32),
            pltpu.VMEM((_NBUF, _WIN, D), table.dtype),
            pltpu.SemaphoreType.DMA((_NBUF,)),
            pltpu.SemaphoreType.DMA((_NBUF,)),
            pltpu.SemaphoreType.DMA((_NBUF,)),
        ],
    )
    def k(table_hbm, idx_hbm, out_hbm, idx_v, rows_v, sem_i, sem_g, sem_w):
        wid = lax.axis_index("s") * mesh.num_cores + lax.axis_index("c")
        lo = wid * base + jnp.minimum(wid, rem)
        hi = lo + base + jnp.where(wid < rem, 1, 0)

        def idx_copy(w, b):
            return pltpu.make_async_copy(
                idx_hbm.at[pl.ds(w * _WIN, _WIN)], idx_v.at[b], sem_i.at[b])

        def gather(b):
            return pltpu.make_async_copy(
                table_hbm.at[idx_v.at[b]], rows_v.at[b], sem_g.at[b])

        def writeout(w, b):
            return pltpu.make_async_copy(
                rows_v.at[b], out_hbm.at[pl.ds(w * _WIN, _WIN)], sem_w.at[b])

        for b in range(_NBUF):
            w = lo + b

            @pl.when(w < hi)
            def _():
                idx_copy(w, b).start()

        @pl.loop(0, tmax)
        def _(t):
            for b in range(_NBUF):
                w = lo + t * _NBUF + b

                @pl.when(w < hi)
                def _():
                    @pl.when(t > 0)
                    def _():
                        writeout(w, b).wait()

                    idx_copy(w, b).wait()
                    gather(b).start()

            for b in range(_NBUF):
                w = lo + t * _NBUF + b

                @pl.when(w < hi)
                def _():
                    gather(b).wait()
                    nw = w + _NBUF

                    @pl.when(nw < hi)
                    def _():
                        idx_copy(nw, b).start()

                    writeout(w, b).start()

        for b in range(_NBUF):
            writeout(lo, b).wait()

    return k(table, idx)


def _k1_body(atom_ref, r_ref):
    r_ref[...] = jnp.sqrt(jnp.abs(atom_ref[...]))


def _k3_body(nblk, m, g_ref, bond_ref, r_ref, wn_ref, bn_ref, au_ref):
    g = g_ref[...].reshape(nblk, m, g_ref.shape[-1])
    bq = bond_ref[...].reshape(nblk, m, bond_ref.shape[-1])
    inv = 1.0 / jnp.sum(bq * bq, axis=-1, keepdims=True)
    den = jnp.maximum(jnp.sum(inv, axis=1, keepdims=True), 1e-12)
    w3 = inv / den
    anw = jnp.sum(g * w3, axis=1)
    x = r_ref[...] * anw
    y = jnp.dot(x, wn_ref[...], preferred_element_type=jnp.float32) + bn_ref[...]
    au_ref[...] = jnp.maximum(y, 0.0)


def _k4_body(nj, d0_ref, d1_ref, s_ref, acc_ref):
    j = pl.program_id(1)

    @pl.when(j == 0)
    def _():
        acc_ref[...] = jnp.zeros_like(acc_ref)

    c0 = jnp.sum(d0_ref[...], axis=0, keepdims=True)
    c1 = jnp.sum(d1_ref[...], axis=0, keepdims=True)
    acc_ref[...] += jnp.concatenate([c0, c1], axis=0)

    @pl.when(j == nj - 1)
    def _():
        s_ref[...] = acc_ref[...].reshape(s_ref.shape)


def _k6_body(nbb, bond_ref, d0_ref, d1_ref, s_ref, wt_ref, wb_ref, we_ref,
             bnte_ref, bedge_ref, out_ref):
    b = pl.program_id(0) // nbb
    s0 = s_ref[pl.ds(2 * b, 1), :]
    s1 = s_ref[pl.ds(2 * b + 1, 1), :]
    r0 = 1.0 / jnp.maximum(s0, 1e-12)
    r1 = 1.0 / jnp.maximum(s1, 1e-12)
    t = jnp.dot(d0_ref[...] * r0, wt_ref[...], preferred_element_type=jnp.float32)
    t += jnp.dot(d1_ref[...] * r1, wb_ref[...], preferred_element_type=jnp.float32)
    y = jnp.tanh(t + bnte_ref[...])
    nb = bond_ref.shape[1]
    fb = bond_ref.shape[-1]
    z = bond_ref[...].reshape(nb * bond_ref.shape[2], fb) + y
    out_ref[...] = (
        jnp.dot(z, we_ref[...], preferred_element_type=jnp.float32) + bedge_ref[...]
    )


def kernel(atom, bond, adj_matrix, adj_matrix_tuple, weight_node, weight_edge,
           weight_node_to_edge, bias_node, bias_edge, bias_node_to_edge):
    B, N, Fa = atom.shape
    M = adj_matrix.shape[-1]
    Fb = bond.shape[-1]
    BN = B * N
    NM = N * M
    TE = B * NM
    f32 = jnp.float32

    atom2 = atom.reshape(BN, Fa)

    blk1 = 2000
    R = pl.pallas_call(
        _k1_body,
        grid=(BN // blk1,),
        in_specs=[pl.BlockSpec((blk1, Fa), lambda i: (i, 0))],
        out_specs=pl.BlockSpec((blk1, Fa), lambda i: (i, 0)),
        out_shape=jax.ShapeDtypeStruct((BN, Fa), f32),
    )(atom2)

    offs = jnp.arange(B, dtype=jnp.int32) * N

    adjg = (adj_matrix + offs[:, None, None]).reshape(B * N * M)
    G = _sc_gather(R, adjg)

    blk3 = 400
    nb3 = N // blk3
    au2 = pl.pallas_call(
        functools.partial(_k3_body, blk3, M),
        grid=(BN // blk3,),
        in_specs=[
            pl.BlockSpec((blk3 * M, Fa), lambda i: (i, 0)),
            pl.BlockSpec((1, blk3, M, Fb),
                         lambda i: (i // nb3, i % nb3, 0, 0)),
            pl.BlockSpec((blk3, Fa), lambda i: (i, 0)),
            pl.BlockSpec((Fa, Fa), lambda i: (0, 0)),
            pl.BlockSpec((1, Fa), lambda i: (0, 0)),
        ],
        out_specs=pl.BlockSpec((blk3, Fa), lambda i: (i, 0)),
        out_shape=jax.ShapeDtypeStruct((BN, Fa), f32),
    )(G, bond, R, weight_node, bias_node.reshape(1, Fa))

    I0 = adj_matrix_tuple[..., 0]
    I1 = adj_matrix_tuple[..., 1]
    I0g = (I0 + offs[:, None]).reshape(TE)
    I1g = (I1 + offs[:, None]).reshape(TE)
    D0 = _sc_gather(au2, I0g)
    D1 = _sc_gather(au2, I1g)

    blk4 = 256
    nj = NM // blk4
    s = pl.pallas_call(
        functools.partial(_k4_body, nj),
        grid=(B, nj),
        in_specs=[
            pl.BlockSpec((blk4, Fa), lambda b, j: (b * nj + j, 0)),
            pl.BlockSpec((blk4, Fa), lambda b, j: (b * nj + j, 0)),
        ],
        out_specs=pl.BlockSpec((1, 2, Fa), lambda b, j: (b, 0, 0)),
        out_shape=jax.ShapeDtypeStruct((B, 2, Fa), f32),
        scratch_shapes=[pltpu.VMEM((2, Fa), f32)],
    )(D0, D1)
    s = s.reshape(2 * B, Fa)

    blk6 = 2000
    nb6 = TE // blk6
    nbb = NM // blk6
    nrow6 = blk6 // M
    nr6 = N // nrow6
    outE = pl.pallas_call(
        functools.partial(_k6_body, nbb),
        grid=(nb6,),
        in_specs=[
            pl.BlockSpec((1, nrow6, M, Fb),
                         lambda i: (i // nr6, i % nr6, 0, 0)),
            pl.BlockSpec((blk6, Fa), lambda i: (i, 0)),
            pl.BlockSpec((blk6, Fa), lambda i: (i, 0)),
            pl.BlockSpec((2 * B, Fa), lambda i: (0, 0)),
            pl.BlockSpec((Fa, Fb), lambda i: (0, 0)),
            pl.BlockSpec((Fa, Fb), lambda i: (0, 0)),
            pl.BlockSpec((Fb, Fb), lambda i: (0, 0)),
            pl.BlockSpec((1, Fb), lambda i: (0, 0)),
            pl.BlockSpec((1, Fb), lambda i: (0, 0)),
        ],
        out_specs=pl.BlockSpec((blk6, Fb), lambda i: (i, 0)),
        out_shape=jax.ShapeDtypeStruct((TE, Fb), f32),
    )(bond, D0, D1, s,
      weight_node_to_edge[:Fa], weight_node_to_edge[Fa:], weight_edge,
      bias_node_to_edge.reshape(1, Fb), bias_edge.reshape(1, Fb))

    return (au2.reshape(B, N, Fa), outE.reshape(B, N, M, Fb))


# speedup vs baseline: 1.4635x; 1.3803x over previous
import functools

import jax
import jax.numpy as jnp
from jax import lax
from jax.experimental import pallas as pl
from jax.experimental.pallas import tpu as pltpu
from jax.experimental.pallas import tpu_sc as plsc

_WIN = 128
_NBUF = 4


def _sc_gather(table, idx):
    T, D = table.shape
    E = idx.shape[0]
    nwin = E // _WIN
    mesh = plsc.VectorSubcoreMesh(core_axis_name="c", subcore_axis_name="s")
    NW = mesh.num_cores * mesh.num_subcores
    base, rem = divmod(nwin, NW)
    tmax = (base + 1 + _NBUF - 1) // _NBUF

    @functools.partial(
        pl.kernel,
        out_type=jax.ShapeDtypeStruct((E, D), table.dtype),
        mesh=mesh,
        scratch_types=[
            pltpu.VMEM((_NBUF, _WIN), jnp.int32),
            pltpu.VMEM((_NBUF, _WIN, D), table.dtype),
            pltpu.SemaphoreType.DMA((_NBUF,)),
            pltpu.SemaphoreType.DMA((_NBUF,)),
            pltpu.SemaphoreType.DMA((_NBUF,)),
        ],
    )
    def k(table_hbm, idx_hbm, out_hbm, idx_v, rows_v, sem_i, sem_g, sem_w):
        wid = lax.axis_index("s") * mesh.num_cores + lax.axis_index("c")
        lo = wid * base + jnp.minimum(wid, rem)
        hi = lo + base + jnp.where(wid < rem, 1, 0)

        def idx_copy(w, b):
            return pltpu.make_async_copy(
                idx_hbm.at[pl.ds(w * _WIN, _WIN)], idx_v.at[b], sem_i.at[b])

        def gather(b):
            return pltpu.make_async_copy(
                table_hbm.at[idx_v.at[b]], rows_v.at[b], sem_g.at[b])

        def writeout(w, b):
            return pltpu.make_async_copy(
                rows_v.at[b], out_hbm.at[pl.ds(w * _WIN, _WIN)], sem_w.at[b])

        for b in range(_NBUF):
            w = lo + b

            @pl.when(w < hi)
            def _():
                idx_copy(w, b).start()

        @pl.loop(0, tmax)
        def _(t):
            for b in range(_NBUF):
                w = lo + t * _NBUF + b

                @pl.when(w < hi)
                def _():
                    @pl.when(t > 0)
                    def _():
                        writeout(w, b).wait()

                    idx_copy(w, b).wait()
                    gather(b).start()

            for b in range(_NBUF):
                w = lo + t * _NBUF + b

                @pl.when(w < hi)
                def _():
                    gather(b).wait()
                    nw = w + _NBUF

                    @pl.when(nw < hi)
                    def _():
                        idx_copy(nw, b).start()

                    writeout(w, b).start()

        for b in range(_NBUF):
            writeout(lo, b).wait()

    return k(table, idx)


def _k1_body(atom_ref, r_ref):
    r_ref[...] = jnp.sqrt(jnp.abs(atom_ref[...]))


def _k3_body(nblk, m, g_ref, bond_ref, r_ref, wn_ref, bn_ref, au_ref):
    g = g_ref[...].reshape(nblk, m, g_ref.shape[-1])
    bq = bond_ref[...].reshape(nblk, m, bond_ref.shape[-1])
    inv = 1.0 / jnp.sum(bq * bq, axis=-1, keepdims=True)
    den = jnp.maximum(jnp.sum(inv, axis=1, keepdims=True), 1e-12)
    w3 = inv / den
    anw = jnp.sum(g * w3, axis=1)
    x = r_ref[...] * anw
    y = jnp.dot(x, wn_ref[...], preferred_element_type=jnp.float32) + bn_ref[...]
    au_ref[...] = jnp.maximum(y, 0.0).reshape(au_ref.shape)


def _k46_body(nbb, bond_ref, d0_ref, d1_ref, wt_ref, wb_ref, we_ref,
              bnte_ref, bedge_ref, out_ref, acc_ref):
    p = pl.program_id(0)
    i = pl.program_id(1)
    b = i // nbb

    @pl.when(p == 0)
    def _():
        @pl.when(i == 0)
        def _():
            acc_ref[...] = jnp.zeros_like(acc_ref)

        c0 = jnp.sum(d0_ref[...], axis=0, keepdims=True)
        c1 = jnp.sum(d1_ref[...], axis=0, keepdims=True)
        acc_ref[pl.ds(2 * b, 1), :] += c0
        acc_ref[pl.ds(2 * b + 1, 1), :] += c1

    @pl.when(p == 1)
    def _():
        s0 = acc_ref[pl.ds(2 * b, 1), :]
        s1 = acc_ref[pl.ds(2 * b + 1, 1), :]
        r0 = 1.0 / jnp.maximum(s0, 1e-12)
        r1 = 1.0 / jnp.maximum(s1, 1e-12)
        t = jnp.dot(d0_ref[...] * r0, wt_ref[...],
                    preferred_element_type=jnp.float32)
        t += jnp.dot(d1_ref[...] * r1, wb_ref[...],
                     preferred_element_type=jnp.float32)
        y = jnp.tanh(t + bnte_ref[...])
        fb = bond_ref.shape[-1]
        z = bond_ref[...].reshape(y.shape[0], fb) + y
        out_ref[...] = (
            jnp.dot(z, we_ref[...], preferred_element_type=jnp.float32)
            + bedge_ref[...]
        ).reshape(out_ref.shape)


def kernel(atom, bond, adj_matrix, adj_matrix_tuple, weight_node, weight_edge,
           weight_node_to_edge, bias_node, bias_edge, bias_node_to_edge):
    B, N, Fa = atom.shape
    M = adj_matrix.shape[-1]
    Fb = bond.shape[-1]
    BN = B * N
    NM = N * M
    TE = B * NM
    f32 = jnp.float32

    atom2 = atom.reshape(BN, Fa)

    blk1 = 2000
    R = pl.pallas_call(
        _k1_body,
        grid=(BN // blk1,),
        in_specs=[pl.BlockSpec((blk1, Fa), lambda i: (i, 0))],
        out_specs=pl.BlockSpec((blk1, Fa), lambda i: (i, 0)),
        out_shape=jax.ShapeDtypeStruct((BN, Fa), f32),
    )(atom2)

    offs = jnp.arange(B, dtype=jnp.int32) * N

    adjg = (adj_matrix + offs[:, None, None]).reshape(B * N * M)
    G = _sc_gather(R, adjg)

    blk3 = 400
    nb3 = N // blk3
    au3 = pl.pallas_call(
        functools.partial(_k3_body, blk3, M),
        grid=(BN // blk3,),
        in_specs=[
            pl.BlockSpec((blk3 * M, Fa), lambda i: (i, 0)),
            pl.BlockSpec((1, blk3, M, Fb),
                         lambda i: (i // nb3, i % nb3, 0, 0)),
            pl.BlockSpec((blk3, Fa), lambda i: (i, 0)),
            pl.BlockSpec((Fa, Fa), lambda i: (0, 0)),
            pl.BlockSpec((1, Fa), lambda i: (0, 0)),
        ],
        out_specs=pl.BlockSpec((1, blk3, Fa), lambda i: (i // nb3, i % nb3, 0)),
        out_shape=jax.ShapeDtypeStruct((B, N, Fa), f32),
    )(G, bond, R, weight_node, bias_node.reshape(1, Fa))
    au2 = au3.reshape(BN, Fa)

    I0 = adj_matrix_tuple[..., 0]
    I1 = adj_matrix_tuple[..., 1]
    I0g = (I0 + offs[:, None]).reshape(TE)
    I1g = (I1 + offs[:, None]).reshape(TE)
    D0 = _sc_gather(au2, I0g)
    D1 = _sc_gather(au2, I1g)

    blk6 = 2000
    nb6 = TE // blk6
    nbb = NM // blk6
    nrow6 = blk6 // M
    nr6 = N // nrow6
    outE = pl.pallas_call(
        functools.partial(_k46_body, nbb),
        grid=(2, nb6),
        in_specs=[
            pl.BlockSpec((1, nrow6, M, Fb),
                         lambda p, i: (i // nr6, i % nr6, 0, 0)),
            pl.BlockSpec((blk6, Fa), lambda p, i: (i, 0)),
            pl.BlockSpec((blk6, Fa), lambda p, i: (i, 0)),
            pl.BlockSpec((Fa, Fb), lambda p, i: (0, 0)),
            pl.BlockSpec((Fa, Fb), lambda p, i: (0, 0)),
            pl.BlockSpec((Fb, Fb), lambda p, i: (0, 0)),
            pl.BlockSpec((1, Fb), lambda p, i: (0, 0)),
            pl.BlockSpec((1, Fb), lambda p, i: (0, 0)),
        ],
        out_specs=pl.BlockSpec((1, nrow6, M, Fb),
                               lambda p, i: (i // nr6, i % nr6, 0, 0)),
        out_shape=jax.ShapeDtypeStruct((B, N, M, Fb), f32),
        scratch_shapes=[pltpu.VMEM((2 * B, Fa), f32)],
    )(bond, D0, D1,
      weight_node_to_edge[:Fa], weight_node_to_edge[Fa:], weight_edge,
      bias_node_to_edge.reshape(1, Fb), bias_edge.reshape(1, Fb))

    return (au3, outE)


# speedup vs baseline: 1.4744x; 1.0075x over previous
import functools

import jax
import jax.numpy as jnp
from jax import lax
from jax.experimental import pallas as pl
from jax.experimental.pallas import tpu as pltpu
from jax.experimental.pallas import tpu_sc as plsc

_WIN = 128
_NBUF = 4


def _sc_gather(table, idx):
    T, D = table.shape
    E = idx.shape[0]
    nwin = E // _WIN
    mesh = plsc.VectorSubcoreMesh(core_axis_name="c", subcore_axis_name="s")
    NW = mesh.num_cores * mesh.num_subcores
    base, rem = divmod(nwin, NW)
    tmax = (base + 1 + _NBUF - 1) // _NBUF

    @functools.partial(
        pl.kernel,
        out_type=jax.ShapeDtypeStruct((E, D), table.dtype),
        mesh=mesh,
        scratch_types=[
            pltpu.VMEM((_NBUF, _WIN), jnp.int32),
            pltpu.VMEM((_NBUF, _WIN, D), table.dtype),
            pltpu.SemaphoreType.DMA((_NBUF,)),
            pltpu.SemaphoreType.DMA((_NBUF,)),
            pltpu.SemaphoreType.DMA((_NBUF,)),
        ],
    )
    def k(table_hbm, idx_hbm, out_hbm, idx_v, rows_v, sem_i, sem_g, sem_w):
        wid = lax.axis_index("s") * mesh.num_cores + lax.axis_index("c")
        lo = wid * base + jnp.minimum(wid, rem)
        hi = lo + base + jnp.where(wid < rem, 1, 0)

        def idx_copy(w, b):
            return pltpu.make_async_copy(
                idx_hbm.at[pl.ds(w * _WIN, _WIN)], idx_v.at[b], sem_i.at[b])

        def gather(b):
            return pltpu.make_async_copy(
                table_hbm.at[idx_v.at[b]], rows_v.at[b], sem_g.at[b])

        def writeout(w, b):
            return pltpu.make_async_copy(
                rows_v.at[b], out_hbm.at[pl.ds(w * _WIN, _WIN)], sem_w.at[b])

        for b in range(_NBUF):
            w = lo + b

            @pl.when(w < hi)
            def _():
                idx_copy(w, b).start()

        @pl.loop(0, tmax)
        def _(t):
            for b in range(_NBUF):
                w = lo + t * _NBUF + b

                @pl.when(w < hi)
                def _():
                    @pl.when(t > 0)
                    def _():
                        writeout(w, b).wait()

                    idx_copy(w, b).wait()
                    gather(b).start()

            for b in range(_NBUF):
                w = lo + t * _NBUF + b

                @pl.when(w < hi)
                def _():
                    gather(b).wait()
                    nw = w + _NBUF

                    @pl.when(nw < hi)
                    def _():
                        idx_copy(nw, b).start()

                    writeout(w, b).start()

        for b in range(_NBUF):
            writeout(lo, b).wait()

    return k(table, idx)


def _k1_body(atom_ref, r_ref):
    r_ref[...] = jnp.sqrt(jnp.abs(atom_ref[...]))


def _k3_body(nblk, m, g_ref, bond_ref, r_ref, wn_ref, bn_ref, au_ref):
    g = g_ref[...].reshape(nblk, m, g_ref.shape[-1])
    bq = bond_ref[...].reshape(nblk, m, bond_ref.shape[-1])
    inv = 1.0 / jnp.sum(bq * bq, axis=-1, keepdims=True)
    den = jnp.maximum(jnp.sum(inv, axis=1, keepdims=True), 1e-12)
    w3 = inv / den
    anw = jnp.sum(g * w3, axis=1)
    x = r_ref[...] * anw
    y = jnp.dot(x, wn_ref[...], preferred_element_type=jnp.float32) + bn_ref[...]
    au_ref[...] = jnp.maximum(y, 0.0).reshape(au_ref.shape)


def _k46_body(nbb, bond_ref, d0_ref, d1_ref, wt_ref, wb_ref, we_ref,
              bnte_ref, bedge_ref, out_ref, acc_ref):
    p = pl.program_id(0)
    i = pl.program_id(1)
    b = i // nbb

    @pl.when(p == 0)
    def _():
        @pl.when(i == 0)
        def _():
            acc_ref[...] = jnp.zeros_like(acc_ref)

        c0 = jnp.sum(d0_ref[...], axis=0, keepdims=True)
        c1 = jnp.sum(d1_ref[...], axis=0, keepdims=True)
        acc_ref[pl.ds(2 * b, 1), :] += c0
        acc_ref[pl.ds(2 * b + 1, 1), :] += c1

    @pl.when(p == 1)
    def _():
        s0 = acc_ref[pl.ds(2 * b, 1), :]
        s1 = acc_ref[pl.ds(2 * b + 1, 1), :]
        r0 = 1.0 / jnp.maximum(s0, 1e-12)
        r1 = 1.0 / jnp.maximum(s1, 1e-12)
        t = jnp.dot(d0_ref[...] * r0, wt_ref[...],
                    preferred_element_type=jnp.float32)
        t += jnp.dot(d1_ref[...] * r1, wb_ref[...],
                     preferred_element_type=jnp.float32)
        y = jnp.tanh(t + bnte_ref[...])
        fb = bond_ref.shape[-1]
        z = bond_ref[...].reshape(y.shape[0], fb) + y
        out_ref[...] = (
            jnp.dot(z, we_ref[...], preferred_element_type=jnp.float32)
            + bedge_ref[...]
        ).reshape(out_ref.shape)


def kernel(atom, bond, adj_matrix, adj_matrix_tuple, weight_node, weight_edge,
           weight_node_to_edge, bias_node, bias_edge, bias_node_to_edge):
    B, N, Fa = atom.shape
    M = adj_matrix.shape[-1]
    Fb = bond.shape[-1]
    BN = B * N
    NM = N * M
    TE = B * NM
    f32 = jnp.float32

    atom2 = atom.reshape(BN, Fa)

    blk1 = 2000
    R = pl.pallas_call(
        _k1_body,
        grid=(BN // blk1,),
        in_specs=[pl.BlockSpec((blk1, Fa), lambda i: (i, 0))],
        out_specs=pl.BlockSpec((blk1, Fa), lambda i: (i, 0)),
        out_shape=jax.ShapeDtypeStruct((BN, Fa), f32),
    )(atom2)

    offs = jnp.arange(B, dtype=jnp.int32) * N

    adjg = (adj_matrix + offs[:, None, None]).reshape(B * N * M)
    G = _sc_gather(R, adjg)

    blk3 = 400
    nb3 = N // blk3
    au3 = pl.pallas_call(
        functools.partial(_k3_body, blk3, M),
        grid=(BN // blk3,),
        in_specs=[
            pl.BlockSpec((blk3 * M, Fa), lambda i: (i, 0)),
            pl.BlockSpec((1, blk3, M, Fb),
                         lambda i: (i // nb3, i % nb3, 0, 0)),
            pl.BlockSpec((blk3, Fa), lambda i: (i, 0)),
            pl.BlockSpec((Fa, Fa), lambda i: (0, 0)),
            pl.BlockSpec((1, Fa), lambda i: (0, 0)),
        ],
        out_specs=pl.BlockSpec((1, blk3, Fa), lambda i: (i // nb3, i % nb3, 0)),
        out_shape=jax.ShapeDtypeStruct((B, N, Fa), f32),
    )(G, bond, R, weight_node, bias_node.reshape(1, Fa))
    au2 = au3.reshape(BN, Fa)

    I0 = adj_matrix_tuple[..., 0]
    I1 = adj_matrix_tuple[..., 1]
    I0g = (I0 + offs[:, None]).reshape(TE)
    I1g = (I1 + offs[:, None]).reshape(TE)
    D = _sc_gather(au2, jnp.concatenate([I0g, I1g]))

    blk6 = 2000
    nb6 = TE // blk6
    nbb = NM // blk6
    nrow6 = blk6 // M
    nr6 = N // nrow6
    outE = pl.pallas_call(
        functools.partial(_k46_body, nbb),
        grid=(2, nb6),
        in_specs=[
            pl.BlockSpec((1, nrow6, M, Fb),
                         lambda p, i: (i // nr6, i % nr6, 0, 0)),
            pl.BlockSpec((blk6, Fa), lambda p, i: (i, 0)),
            pl.BlockSpec((blk6, Fa), lambda p, i: (i + nb6, 0)),
            pl.BlockSpec((Fa, Fb), lambda p, i: (0, 0)),
            pl.BlockSpec((Fa, Fb), lambda p, i: (0, 0)),
            pl.BlockSpec((Fb, Fb), lambda p, i: (0, 0)),
            pl.BlockSpec((1, Fb), lambda p, i: (0, 0)),
            pl.BlockSpec((1, Fb), lambda p, i: (0, 0)),
        ],
        out_specs=pl.BlockSpec((1, nrow6, M, Fb),
                               lambda p, i: (i // nr6, i % nr6, 0, 0)),
        out_shape=jax.ShapeDtypeStruct((B, N, M, Fb), f32),
        scratch_shapes=[pltpu.VMEM((2 * B, Fa), f32)],
    )(bond, D, D,
      weight_node_to_edge[:Fa], weight_node_to_edge[Fa:], weight_edge,
      bias_node_to_edge.reshape(1, Fb), bias_edge.reshape(1, Fb))

    return (au3, outE)


# speedup vs baseline: 1.4761x; 1.0012x over previous
import functools

import jax
import jax.numpy as jnp
from jax import lax
from jax.experimental import pallas as pl
from jax.experimental.pallas import tpu as pltpu
from jax.experimental.pallas import tpu_sc as plsc

_WIN = 128
_NBUF = 6


def _sc_gather(table, idx):
    T, D = table.shape
    E = idx.shape[0]
    nwin = E // _WIN
    mesh = plsc.VectorSubcoreMesh(core_axis_name="c", subcore_axis_name="s")
    NW = mesh.num_cores * mesh.num_subcores
    base, rem = divmod(nwin, NW)
    tmax = (base + 1 + _NBUF - 1) // _NBUF

    @functools.partial(
        pl.kernel,
        out_type=jax.ShapeDtypeStruct((E, D), table.dtype),
        mesh=mesh,
        scratch_types=[
            pltpu.VMEM((_NBUF, _WIN), jnp.int32),
            pltpu.VMEM((_NBUF, _WIN, D), table.dtype),
            pltpu.SemaphoreType.DMA((_NBUF,)),
            pltpu.SemaphoreType.DMA((_NBUF,)),
            pltpu.SemaphoreType.DMA((_NBUF,)),
        ],
    )
    def k(table_hbm, idx_hbm, out_hbm, idx_v, rows_v, sem_i, sem_g, sem_w):
        wid = lax.axis_index("s") * mesh.num_cores + lax.axis_index("c")
        lo = wid * base + jnp.minimum(wid, rem)
        hi = lo + base + jnp.where(wid < rem, 1, 0)

        def idx_copy(w, b):
            return pltpu.make_async_copy(
                idx_hbm.at[pl.ds(w * _WIN, _WIN)], idx_v.at[b], sem_i.at[b])

        def gather(b):
            return pltpu.make_async_copy(
                table_hbm.at[idx_v.at[b]], rows_v.at[b], sem_g.at[b])

        def writeout(w, b):
            return pltpu.make_async_copy(
                rows_v.at[b], out_hbm.at[pl.ds(w * _WIN, _WIN)], sem_w.at[b])

        for b in range(_NBUF):
            w = lo + b

            @pl.when(w < hi)
            def _():
                idx_copy(w, b).start()

        @pl.loop(0, tmax)
        def _(t):
            for b in range(_NBUF):
                w = lo + t * _NBUF + b

                @pl.when(w < hi)
                def _():
                    @pl.when(t > 0)
                    def _():
                        writeout(w, b).wait()

                    idx_copy(w, b).wait()
                    gather(b).start()

            for b in range(_NBUF):
                w = lo + t * _NBUF + b

                @pl.when(w < hi)
                def _():
                    gather(b).wait()
                    nw = w + _NBUF

                    @pl.when(nw < hi)
                    def _():
                        idx_copy(nw, b).start()

                    writeout(w, b).start()

        for b in range(_NBUF):
            writeout(lo, b).wait()

    return k(table, idx)


def _k1_body(atom_ref, r_ref):
    r_ref[...] = jnp.sqrt(jnp.abs(atom_ref[...]))


def _k3_body(nblk, m, g_ref, bond_ref, r_ref, wn_ref, bn_ref, au_ref):
    g = g_ref[...].reshape(nblk, m, g_ref.shape[-1])
    bq = bond_ref[...].reshape(nblk, m, bond_ref.shape[-1])
    inv = 1.0 / jnp.sum(bq * bq, axis=-1, keepdims=True)
    den = jnp.maximum(jnp.sum(inv, axis=1, keepdims=True), 1e-12)
    w3 = inv / den
    anw = jnp.sum(g * w3, axis=1)
    x = r_ref[...] * anw
    y = jnp.dot(x, wn_ref[...], preferred_element_type=jnp.float32) + bn_ref[...]
    au_ref[...] = jnp.maximum(y, 0.0).reshape(au_ref.shape)


def _k46_body(nbb, bond_ref, d0_ref, d1_ref, wt_ref, wb_ref, we_ref,
              bnte_ref, bedge_ref, out_ref, acc_ref):
    p = pl.program_id(0)
    i = pl.program_id(1)
    b = i // nbb

    @pl.when(p == 0)
    def _():
        @pl.when(i == 0)
        def _():
            acc_ref[...] = jnp.zeros_like(acc_ref)

        c0 = jnp.sum(d0_ref[...], axis=0, keepdims=True)
        c1 = jnp.sum(d1_ref[...], axis=0, keepdims=True)
        acc_ref[pl.ds(2 * b, 1), :] += c0
        acc_ref[pl.ds(2 * b + 1, 1), :] += c1

    @pl.when(p == 1)
    def _():
        s0 = acc_ref[pl.ds(2 * b, 1), :]
        s1 = acc_ref[pl.ds(2 * b + 1, 1), :]
        r0 = 1.0 / jnp.maximum(s0, 1e-12)
        r1 = 1.0 / jnp.maximum(s1, 1e-12)
        t = jnp.dot(d0_ref[...] * r0, wt_ref[...],
                    preferred_element_type=jnp.float32)
        t += jnp.dot(d1_ref[...] * r1, wb_ref[...],
                     preferred_element_type=jnp.float32)
        y = jnp.tanh(t + bnte_ref[...])
        fb = bond_ref.shape[-1]
        z = bond_ref[...].reshape(y.shape[0], fb) + y
        out_ref[...] = (
            jnp.dot(z, we_ref[...], preferred_element_type=jnp.float32)
            + bedge_ref[...]
        ).reshape(out_ref.shape)


def kernel(atom, bond, adj_matrix, adj_matrix_tuple, weight_node, weight_edge,
           weight_node_to_edge, bias_node, bias_edge, bias_node_to_edge):
    B, N, Fa = atom.shape
    M = adj_matrix.shape[-1]
    Fb = bond.shape[-1]
    BN = B * N
    NM = N * M
    TE = B * NM
    f32 = jnp.float32

    atom2 = atom.reshape(BN, Fa)

    blk1 = 2000
    R = pl.pallas_call(
        _k1_body,
        grid=(BN // blk1,),
        in_specs=[pl.BlockSpec((blk1, Fa), lambda i: (i, 0))],
        out_specs=pl.BlockSpec((blk1, Fa), lambda i: (i, 0)),
        out_shape=jax.ShapeDtypeStruct((BN, Fa), f32),
    )(atom2)

    offs = jnp.arange(B, dtype=jnp.int32) * N

    adjg = (adj_matrix + offs[:, None, None]).reshape(B * N * M)
    G = _sc_gather(R, adjg)

    blk3 = 400
    nb3 = N // blk3
    au3 = pl.pallas_call(
        functools.partial(_k3_body, blk3, M),
        grid=(BN // blk3,),
        in_specs=[
            pl.BlockSpec((blk3 * M, Fa), lambda i: (i, 0)),
            pl.BlockSpec((1, blk3, M, Fb),
                         lambda i: (i // nb3, i % nb3, 0, 0)),
            pl.BlockSpec((blk3, Fa), lambda i: (i, 0)),
            pl.BlockSpec((Fa, Fa), lambda i: (0, 0)),
            pl.BlockSpec((1, Fa), lambda i: (0, 0)),
        ],
        out_specs=pl.BlockSpec((1, blk3, Fa), lambda i: (i // nb3, i % nb3, 0)),
        out_shape=jax.ShapeDtypeStruct((B, N, Fa), f32),
    )(G, bond, R, weight_node, bias_node.reshape(1, Fa))
    au2 = au3.reshape(BN, Fa)

    I0 = adj_matrix_tuple[..., 0]
    I1 = adj_matrix_tuple[..., 1]
    I0g = (I0 + offs[:, None]).reshape(TE)
    I1g = (I1 + offs[:, None]).reshape(TE)
    D = _sc_gather(au2, jnp.concatenate([I0g, I1g]))

    blk6 = 2000
    nb6 = TE // blk6
    nbb = NM // blk6
    nrow6 = blk6 // M
    nr6 = N // nrow6
    outE = pl.pallas_call(
        functools.partial(_k46_body, nbb),
        grid=(2, nb6),
        in_specs=[
            pl.BlockSpec((1, nrow6, M, Fb),
                         lambda p, i: (i // nr6, i % nr6, 0, 0)),
            pl.BlockSpec((blk6, Fa), lambda p, i: (i, 0)),
            pl.BlockSpec((blk6, Fa), lambda p, i: (i + nb6, 0)),
            pl.BlockSpec((Fa, Fb), lambda p, i: (0, 0)),
            pl.BlockSpec((Fa, Fb), lambda p, i: (0, 0)),
            pl.BlockSpec((Fb, Fb), lambda p, i: (0, 0)),
            pl.BlockSpec((1, Fb), lambda p, i: (0, 0)),
            pl.BlockSpec((1, Fb), lambda p, i: (0, 0)),
        ],
        out_specs=pl.BlockSpec((1, nrow6, M, Fb),
                               lambda p, i: (i // nr6, i % nr6, 0, 0)),
        out_shape=jax.ShapeDtypeStruct((B, N, M, Fb), f32),
        scratch_shapes=[pltpu.VMEM((2 * B, Fa), f32)],
    )(bond, D, D,
      weight_node_to_edge[:Fa], weight_node_to_edge[Fa:], weight_edge,
      bias_node_to_edge.reshape(1, Fb), bias_edge.reshape(1, Fb))

    return (au3, outE)
